# baseline (speedup 1.0000x reference)
import functools

import numpy as np
import jax
import jax.numpy as jnp
from jax import lax
from jax.experimental import pallas as pl
from jax.experimental.pallas import tpu as pltpu
from jax.experimental.pallas import tpu_sc as plsc

_T = 8192
_M = 256
_MC = 64
_C = 64
_COEF = 2.0 * np.pi / _M + 1e-4
_SENT = 2**30

_ANG = 2.0 * np.pi * np.arange(_M) / _M
_DIRS = np.stack([np.cos(_ANG), np.sin(_ANG)]).astype(np.float32)

_NC = 2
_NS = 16
_L = 16
_NTAB = 8


_DNT = (((1,), (1,)), ((), ()))


def _a_body(emb_ref, wq4_ref, wtab_ref, wdpt_ref, wdst_ref,
            tabs_ref, idx8_ref, qw_ref):
    emb = emb_ref[...]
    iota128 = lax.broadcasted_iota(jnp.int32, (1, 128), 1)

    qw_ref[...] = jnp.dot(emb, wq4_ref[...])
    tabs = lax.dot_general(wtab_ref[...], emb, _DNT)
    tabs_ref[...] = tabs

    def mask_of(wdt_ref, krow):
        k0 = tabs[krow:krow + 1, :]
        k1 = tabs[krow + 1:krow + 2, :]
        r2 = jnp.max(k0 * k0 + k1 * k1, axis=1, keepdims=True)
        delta = jnp.sqrt(r2) * _COEF
        mask_row = None
        for c in range(_M // _MC):
            pc = lax.dot_general(
                wdt_ref[c * _MC:(c + 1) * _MC, :], emb, _DNT)
            mxc = jnp.max(pc, axis=1, keepdims=True)
            hit = (pc >= mxc - delta).astype(jnp.float32)
            part = jnp.max(hit, axis=0, keepdims=True)
            mask_row = part if mask_row is None else jnp.maximum(mask_row, part)
        rows = [
            jnp.where(mask_row[0:1, r * 128:(r + 1) * 128] > 0.0,
                      iota128 + r * 128, _SENT)
            for r in range(_T // 128)
        ]
        return jnp.concatenate(rows, axis=0)

    mip = mask_of(wdpt_ref, 0)
    mis = mask_of(wdst_ref, 4)

    outp, outs = [], []
    c0p = c0s = None
    for _ in range(_C):
        curp = jnp.min(jnp.min(mip, axis=1, keepdims=True), axis=0,
                       keepdims=True)
        curs = jnp.min(jnp.min(mis, axis=1, keepdims=True), axis=0,
                       keepdims=True)
        if c0p is None:
            c0p, c0s = curp, curs
        outp.append(jnp.where(curp < _SENT, curp, c0p))
        outs.append(jnp.where(curs < _SENT, curs, c0s))
        mip = jnp.where(mip == curp, _SENT, mip)
        mis = jnp.where(mis == curs, _SENT, mis)
    cp = jnp.concatenate(outp, axis=1)
    cs = jnp.concatenate(outs, axis=1)
    idx8_ref[...] = jnp.concatenate(
        [cp, cp, cp, cp, cs, cs, cs, cp], axis=0)


def _stage_a(emb, wq4, wtab, wdpt, wdst):
    sd = jax.ShapeDtypeStruct
    return pl.pallas_call(
        _a_body,
        out_shape=[
            sd((_NTAB, _T), jnp.float32),
            sd((_NTAB, _C), jnp.int32),
            sd((_T, 4), jnp.float32),
        ],
    )(emb, wq4, wtab, wdpt, wdst)


def _b_body(tab_hbm, idx_hbm, out_hbm, tab_v, idx_v, out_v):
    wid = lax.axis_index("s") * _NC + lax.axis_index("c")
    tbase = (wid // 4) * _T
    pltpu.sync_copy(tab_hbm.at[pl.ds(tbase, _T)], tab_v)
    pltpu.sync_copy(idx_hbm.at[pl.ds(wid * _L, _L)], idx_v)
    out_v[...] = plsc.load_gather(tab_v, [idx_v[...]])
    pltpu.sync_copy(out_v, out_hbm.at[pl.ds(wid * _L, _L)])


@functools.cache
def _stage_b():
    return functools.partial(
        pl.kernel,
        mesh=plsc.VectorSubcoreMesh(core_axis_name="c", subcore_axis_name="s"),
        compiler_params=pltpu.CompilerParams(
            use_tc_tiling_on_sc=False, needs_layout_passes=False),
        out_type=jax.ShapeDtypeStruct((_NTAB * _C,), jnp.float32),
        scratch_types=[
            pltpu.VMEM((_T,), jnp.float32),
            pltpu.VMEM((_L,), jnp.int32),
            pltpu.VMEM((_L,), jnp.float32),
        ],
    )(_b_body)


def _c_body(qw_ref, g_ref, fo_ref, fa_ref, fv_ref):
    iota = lax.broadcasted_iota(jnp.int32, (_T, _C), 1)

    def head(qcol, krow):
        q2 = qw_ref[:, qcol:qcol + 2]
        kc = g_ref[krow:krow + 2, :]
        s = jnp.dot(q2, kc)
        m = jnp.max(s, axis=1, keepdims=True)
        pos = jnp.min(jnp.where(s == m, iota, _C), axis=1, keepdims=True)
        return iota == pos

    def pick(oh, prow):
        return jnp.sum(jnp.where(oh, g_ref[prow:prow + 1, :], 0.0),
                       axis=1, keepdims=True)

    ohp = head(0, 0)
    fo_ref[...] = pick(ohp, 2)
    fa_ref[...] = pick(ohp, 3)
    ohs = head(2, 4)
    fv_ref[...] = pick(ohs, 6)


def _stage_c(qw, g):
    sd = jax.ShapeDtypeStruct
    return pl.pallas_call(
        _c_body,
        out_shape=[sd((_T, 1), jnp.float32)] * 3,
    )(qw, g)


def kernel(embeddings, WQ_prog, WK_prog, WV_op, WV_arg, WQ_stack, WK_stack, WV_stack):
    dirs = jnp.asarray(_DIRS)
    wdpt = (WK_prog.T @ dirs).T
    wdst = (WK_stack.T @ dirs).T
    wq4 = jnp.concatenate([WQ_prog.T, WQ_stack.T], axis=1)
    wtab = jnp.concatenate([WK_prog, WV_op, WV_arg, WK_stack, WV_stack,
                            WK_prog[0:1]], axis=0)

    tabs, idx8, qw = _stage_a(embeddings, wq4, wtab, wdpt, wdst)
    out_all = _stage_b()(tabs.reshape(_NTAB * _T), idx8.reshape(_NTAB * _C))
    fo, fa, fv = _stage_c(qw, out_all.reshape(_NTAB, _C))
    return (fo.reshape(_T), fa.reshape(_T), fv.reshape(_T))


# speedup vs baseline: 1.0316x; 1.0316x over previous
import functools

import numpy as np
import jax
import jax.numpy as jnp
from jax import lax
from jax.experimental import pallas as pl
from jax.experimental.pallas import tpu as pltpu
from jax.experimental.pallas import tpu_sc as plsc

_T = 8192
_M = 256
_MC = 64
_C = 64
_COEF = 2.0 * np.pi / _M + 1e-4
_SENT = 2**30

_ANG = 2.0 * np.pi * np.arange(_M) / _M
_DIRS = np.stack([np.cos(_ANG), np.sin(_ANG)]).astype(np.float32)

_NC = 2
_NS = 16
_L = 16
_NTAB = 8


def _a_body(emb_ref, embt_ref, wq4_ref, wtab_ref, wdpt_ref, wdst_ref,
            tabs_ref, idx8_ref, qw_ref):
    emb = emb_ref[...]
    embt = embt_ref[...]
    iota128 = lax.broadcasted_iota(jnp.int32, (1, 128), 1)

    qw_ref[...] = jnp.dot(emb, wq4_ref[...])
    tabs = jnp.dot(wtab_ref[...], embt)
    tabs_ref[...] = tabs

    def mask_of(wdt_ref, krow):
        k0 = tabs[krow:krow + 1, :]
        k1 = tabs[krow + 1:krow + 2, :]
        r2 = jnp.max(k0 * k0 + k1 * k1, axis=1, keepdims=True)
        delta = jnp.sqrt(r2) * _COEF
        mask_row = None
        for c in range(_M // _MC):
            pc = jnp.dot(wdt_ref[c * _MC:(c + 1) * _MC, :], embt)
            mxc = jnp.max(pc, axis=1, keepdims=True)
            hit = (pc >= mxc - delta).astype(jnp.float32)
            part = jnp.max(hit, axis=0, keepdims=True)
            mask_row = part if mask_row is None else jnp.maximum(mask_row, part)
        rows = [
            jnp.where(mask_row[0:1, r * 128:(r + 1) * 128] > 0.0,
                      iota128 + r * 128, _SENT)
            for r in range(_T // 128)
        ]
        return jnp.concatenate(rows, axis=0)

    mip = mask_of(wdpt_ref, 0)
    mis = mask_of(wdst_ref, 4)

    outp, outs = [], []
    c0p = c0s = None
    for _ in range(_C):
        curp = jnp.min(jnp.min(mip, axis=1, keepdims=True), axis=0,
                       keepdims=True)
        curs = jnp.min(jnp.min(mis, axis=1, keepdims=True), axis=0,
                       keepdims=True)
        if c0p is None:
            c0p, c0s = curp, curs
        outp.append(jnp.where(curp < _SENT, curp, c0p))
        outs.append(jnp.where(curs < _SENT, curs, c0s))
        mip = jnp.where(mip == curp, _SENT, mip)
        mis = jnp.where(mis == curs, _SENT, mis)
    cp = jnp.concatenate(outp, axis=1)
    cs = jnp.concatenate(outs, axis=1)
    idx8_ref[...] = jnp.concatenate(
        [cp, cp, cp, cp, cs, cs, cs, cp], axis=0)


def _stage_a(emb, embt, wq4, wtab, wdpt, wdst):
    sd = jax.ShapeDtypeStruct
    return pl.pallas_call(
        _a_body,
        out_shape=[
            sd((_NTAB, _T), jnp.float32),
            sd((_NTAB, _C), jnp.int32),
            sd((_T, 4), jnp.float32),
        ],
    )(emb, embt, wq4, wtab, wdpt, wdst)


def _b_body(tab_hbm, idx_hbm, out_hbm, tab_v, idx_v, out_v):
    wid = lax.axis_index("s") * _NC + lax.axis_index("c")
    tbase = (wid // 4) * _T
    pltpu.sync_copy(tab_hbm.at[pl.ds(tbase, _T)], tab_v)
    pltpu.sync_copy(idx_hbm.at[pl.ds(wid * _L, _L)], idx_v)
    out_v[...] = plsc.load_gather(tab_v, [idx_v[...]])
    pltpu.sync_copy(out_v, out_hbm.at[pl.ds(wid * _L, _L)])


@functools.cache
def _stage_b():
    return functools.partial(
        pl.kernel,
        mesh=plsc.VectorSubcoreMesh(core_axis_name="c", subcore_axis_name="s"),
        compiler_params=pltpu.CompilerParams(
            use_tc_tiling_on_sc=False, needs_layout_passes=False),
        out_type=jax.ShapeDtypeStruct((_NTAB * _C,), jnp.float32),
        scratch_types=[
            pltpu.VMEM((_T,), jnp.float32),
            pltpu.VMEM((_L,), jnp.int32),
            pltpu.VMEM((_L,), jnp.float32),
        ],
    )(_b_body)


def _c_body(qw_ref, g_ref, fo_ref, fa_ref, fv_ref):
    iota = lax.broadcasted_iota(jnp.int32, (_T, _C), 1)

    def head(qcol, krow):
        q2 = qw_ref[:, qcol:qcol + 2]
        kc = g_ref[krow:krow + 2, :]
        s = jnp.dot(q2, kc)
        m = jnp.max(s, axis=1, keepdims=True)
        pos = jnp.min(jnp.where(s == m, iota, _C), axis=1, keepdims=True)
        return iota == pos

    def pick(oh, prow):
        return jnp.sum(jnp.where(oh, g_ref[prow:prow + 1, :], 0.0),
                       axis=1, keepdims=True)

    ohp = head(0, 0)
    fo_ref[...] = pick(ohp, 2)
    fa_ref[...] = pick(ohp, 3)
    ohs = head(2, 4)
    fv_ref[...] = pick(ohs, 6)


def _stage_c(qw, g):
    sd = jax.ShapeDtypeStruct
    return pl.pallas_call(
        _c_body,
        out_shape=[sd((_T, 1), jnp.float32)] * 3,
    )(qw, g)


def kernel(embeddings, WQ_prog, WK_prog, WV_op, WV_arg, WQ_stack, WK_stack, WV_stack):
    dirs = jnp.asarray(_DIRS)
    wdpt = (WK_prog.T @ dirs).T
    wdst = (WK_stack.T @ dirs).T
    wq4 = jnp.concatenate([WQ_prog.T, WQ_stack.T], axis=1)
    wtab = jnp.concatenate([WK_prog, WV_op, WV_arg, WK_stack, WV_stack,
                            WK_prog[0:1]], axis=0)

    tabs, idx8, qw = _stage_a(embeddings, embeddings.T, wq4, wtab, wdpt, wdst)
    out_all = _stage_b()(tabs.reshape(_NTAB * _T), idx8.reshape(_NTAB * _C))
    fo, fa, fv = _stage_c(qw, out_all.reshape(_NTAB, _C))
    return (fo.reshape(_T), fa.reshape(_T), fv.reshape(_T))
